# baseline (speedup 1.0000x reference)
import functools

import jax
import jax.numpy as jnp
from jax import lax
from jax.experimental import pallas as pl
from jax.experimental.pallas import tpu as pltpu, tpu_sc as plsc

B = 16384
EMBED = 64
NC, NS, L = 2, 16, 16
NW = NC * NS
BPW = B // NW
CHUNK = 128
NCHUNK = BPW // CHUNK
NGROUP = BPW // L


def _rsqrt(x):
    i = plsc.bitcast(x, jnp.int32)
    i = jnp.int32(0x5F3759DF) - (i >> 1)
    y = plsc.bitcast(i, jnp.float32)
    for _ in range(3):
        y = y * (1.5 - 0.5 * x * y * y)
    return y


def _body(fidx2, tidx2, ftab, ttab, wvec_h, bvec_h, out_h,
          idx_f, idx_t, rows_f, rows_t, out_v, wb_v, sem):
    wid = lax.axis_index("s") * NC + lax.axis_index("c")
    base = wid * BPW

    pltpu.sync_copy(fidx2.at[pl.ds(wid * NCHUNK, NCHUNK)], idx_f)
    pltpu.sync_copy(tidx2.at[pl.ds(wid * NCHUNK, NCHUNK)], idx_t)
    pltpu.sync_copy(wvec_h, wb_v.at[0])
    pltpu.sync_copy(bvec_h, wb_v.at[1])

    copies = []
    for j in range(NCHUNK):
        copies.append(pltpu.async_copy(
            ftab.at[idx_f.at[j]], rows_f.at[pl.ds(j * CHUNK, CHUNK)], sem))
        copies.append(pltpu.async_copy(
            ttab.at[idx_t.at[j]], rows_t.at[pl.ds(j * CHUNK, CHUNK)], sem))
    for c in copies:
        c.wait()

    w = wb_v[0]
    b = wb_v[1]
    eps = jnp.full((L,), 1e-12, jnp.float32)
    lane = lax.iota(jnp.int32, L)
    zero = jnp.zeros((L,), jnp.float32)

    for g in range(NGROUP):
        rows = lane + (g * L)

        def body(d, carry):
            acc_d, acc_a, acc_b = carry
            col = jnp.full((L,), d, jnp.int32)
            gf = plsc.load_gather(rows_f, [rows, col])
            gt = plsc.load_gather(rows_t, [rows, col])
            return (acc_d + gf * gt, acc_a + gf * gf, acc_b + gt * gt)

        acc_d, acc_a, acc_b = lax.fori_loop(
            0, EMBED, body, (zero, zero, zero), unroll=8)

        inv = _rsqrt(jnp.maximum(acc_a, eps) * jnp.maximum(acc_b, eps))
        z = acc_d * inv * w + b
        out_v[pl.ds(g * L, L)] = 1.0 / (1.0 + jnp.exp(-z))

    pltpu.sync_copy(out_v, out_h.at[pl.ds(base, BPW)])


@functools.partial(jax.jit, static_argnames=())
def kernel(file, test, file_table, test_table, dense_w, dense_b):
    fidx2 = file.astype(jnp.int32).reshape(NW * NCHUNK, CHUNK)
    tidx2 = test.astype(jnp.int32).reshape(NW * NCHUNK, CHUNK)
    wvec = jnp.broadcast_to(dense_w.reshape(1), (L,)).astype(jnp.float32)
    bvec = jnp.broadcast_to(dense_b.reshape(1), (L,)).astype(jnp.float32)

    mesh = plsc.VectorSubcoreMesh(core_axis_name="c", subcore_axis_name="s")
    out = pl.kernel(
        _body,
        out_type=jax.ShapeDtypeStruct((B,), jnp.float32),
        mesh=mesh,
        compiler_params=pltpu.CompilerParams(
            needs_layout_passes=False, use_tc_tiling_on_sc=False),
        scratch_types=[
            pltpu.VMEM((NCHUNK, CHUNK), jnp.int32),
            pltpu.VMEM((NCHUNK, CHUNK), jnp.int32),
            pltpu.VMEM((BPW, EMBED), jnp.float32),
            pltpu.VMEM((BPW, EMBED), jnp.float32),
            pltpu.VMEM((BPW,), jnp.float32),
            pltpu.VMEM((2, L), jnp.float32),
            pltpu.SemaphoreType.DMA,
        ],
    )(fidx2, tidx2, file_table, test_table, wvec, bvec)
    return out.reshape(B, 1)


# speedup vs baseline: 1.9874x; 1.9874x over previous
import jax
import jax.numpy as jnp
from jax import lax
from jax.experimental import pallas as pl
from jax.experimental.pallas import tpu as pltpu, tpu_sc as plsc

B = 16384
EMBED = 64
SUB = 8
NC, NS, L = 2, 16, 16
NW = NC * NS
BPW = B // NW
CHUNK = 32
NCHUNK = BPW // CHUNK
GPC = CHUNK // L


def _rsqrt(x):
    i = plsc.bitcast(x, jnp.int32)
    i = jnp.int32(0x5F3759DF) - (i >> 1)
    y = plsc.bitcast(i, jnp.float32)
    for _ in range(3):
        y = y * (1.5 - 0.5 * x * y * y)
    return y


def _body(fidx2, tidx2, ftab, ttab, wvec_h, bvec_h, out_h,
          idx_fv, idx_tv, buf_f, buf_t, out_v, wb_v, semf, semt):
    wid = lax.axis_index("s") * NC + lax.axis_index("c")
    base = wid * BPW

    pltpu.sync_copy(fidx2.at[pl.ds(base, BPW)], idx_fv)
    pltpu.sync_copy(tidx2.at[pl.ds(base, BPW)], idx_tv)
    pltpu.sync_copy(wvec_h, wb_v.at[0])
    pltpu.sync_copy(bvec_h, wb_v.at[1])

    w = wb_v[0]
    b = wb_v[1]
    eps = jnp.full((L,), 1e-12, jnp.float32)
    lane = lax.iota(jnp.int32, L)
    zero = jnp.zeros((L,), jnp.float32)

    def fire(c):
        for gl in range(GPC):
            tv_f = idx_fv[pl.ds(c * CHUNK + gl * L, L)] >> 3
            tv_t = idx_tv[pl.ds(c * CHUNK + gl * L, L)] >> 3
            for k in range(L):
                i = gl * L + k
                pltpu.async_copy(ftab.at[tv_f[k]], buf_f.at[i], semf)
                pltpu.async_copy(ttab.at[tv_t[k]], buf_t.at[i], semt)

    def drain():
        pltpu.make_async_copy(ftab.at[pl.ds(0, CHUNK)], buf_f, semf).wait()
        pltpu.make_async_copy(ttab.at[pl.ds(0, CHUNK)], buf_t, semt).wait()

    def chunk(c, _):
        fire(c)
        drain()

        for gl in range(GPC):
            off = c * CHUNK + gl * L
            iv_f = idx_fv[pl.ds(off, L)]
            iv_t = idx_tv[pl.ds(off, L)]
            sub_f = iv_f & 7
            sub_t = iv_t & 7
            rows = lane + (gl * L)

            def body(d, carry, sub_f=sub_f, sub_t=sub_t, rows=rows):
                acc_d, acc_a, acc_b = carry
                col = jnp.full((L,), d, jnp.int32)
                gf = plsc.load_gather(buf_f, [rows, sub_f, col])
                gt = plsc.load_gather(buf_t, [rows, sub_t, col])
                return (acc_d + gf * gt, acc_a + gf * gf, acc_b + gt * gt)

            acc_d, acc_a, acc_b = lax.fori_loop(
                0, EMBED, body, (zero, zero, zero), unroll=8)

            inv = _rsqrt(jnp.maximum(acc_a, eps) * jnp.maximum(acc_b, eps))
            z = acc_d * inv * w + b
            out_v[pl.ds(off, L)] = 1.0 / (1.0 + jnp.exp(-z))
        return 0

    lax.fori_loop(0, NCHUNK, chunk, 0)

    pltpu.sync_copy(out_v, out_h.at[pl.ds(base, BPW)])


@jax.jit
def kernel(file, test, file_table, test_table, dense_w, dense_b):
    fidx = file.astype(jnp.int32)
    tidx = test.astype(jnp.int32)
    ft3 = file_table.reshape(file_table.shape[0] // SUB, SUB, EMBED)
    tt3 = test_table.reshape(test_table.shape[0] // SUB, SUB, EMBED)
    wvec = jnp.broadcast_to(dense_w.reshape(1), (L,)).astype(jnp.float32)
    bvec = jnp.broadcast_to(dense_b.reshape(1), (L,)).astype(jnp.float32)

    mesh = plsc.VectorSubcoreMesh(core_axis_name="c", subcore_axis_name="s")
    out = pl.kernel(
        _body,
        out_type=jax.ShapeDtypeStruct((B,), jnp.float32),
        mesh=mesh,
        compiler_params=pltpu.CompilerParams(needs_layout_passes=False),
        scratch_types=[
            pltpu.VMEM((BPW,), jnp.int32),
            pltpu.VMEM((BPW,), jnp.int32),
            pltpu.VMEM((CHUNK, SUB, EMBED), jnp.float32),
            pltpu.VMEM((CHUNK, SUB, EMBED), jnp.float32),
            pltpu.VMEM((BPW,), jnp.float32),
            pltpu.VMEM((2, L), jnp.float32),
            pltpu.SemaphoreType.DMA,
            pltpu.SemaphoreType.DMA,
        ],
    )(fidx, tidx, ft3, tt3, wvec, bvec)
    return out.reshape(B, 1)
